# Initial kernel scaffold; baseline (speedup 1.0000x reference)
import functools

import jax
import jax.numpy as jnp
import numpy as np
from jax import lax
from jax.experimental import pallas as pl
from jax.experimental.pallas import tpu as pltpu
from jax.experimental.pallas import tpu_sc as plsc

H = 224
W = 224
V = H * W
E = (H - 1) * W + H * (W - 1)
CE = 2048
E_PAD = ((E + CE - 1) // CE) * CE
OUTCH = 4096
FULL_FLUSHES = (V - 1) // OUTCH
TAIL_ROWS = (V - 1) - FULL_FLUSHES * OUTCH
TAIL_COPY_W = 2 * (TAIL_ROWS + 1)
C_BLK = 16


def _edge_uv_const():
    r = np.arange(V, dtype=np.int64).reshape(H, W)
    u = np.concatenate([r[:-1].reshape(-1), r[:, :-1].reshape(-1)])
    v = np.concatenate([r[1:].reshape(-1), r[:, 1:].reshape(-1)])
    uv = (u << 16) | v
    return uv.astype(np.uint32).view(np.int32)


_UV = _edge_uv_const()


def _weights_body(x_ref, wv_ref, wh_ref, accv, acch):
    c_idx = pl.program_id(1)
    for c in range(C_BLK):
        xc = x_ref[0, c]
        dv = xc[1:, :] - xc[:-1, :]
        dh = xc[:, 1:] - xc[:, :-1]
        if c == 0:
            accv[...] = jnp.where(c_idx == 0, dv * dv, accv[...] + dv * dv)
            acch[...] = jnp.where(c_idx == 0, dh * dh, acch[...] + dh * dh)
        else:
            accv[...] = accv[...] + dv * dv
            acch[...] = acch[...] + dh * dh

    @pl.when(c_idx == pl.num_programs(1) - 1)
    def _():
        wv_ref[0] = jnp.sqrt(accv[...]) + 1.0
        wh_ref[0] = jnp.sqrt(acch[...]) + 1.0


def _edge_weights(guide):
    B, C, _, _ = guide.shape
    nc = C // C_BLK
    return pl.pallas_call(
        _weights_body,
        grid=(B, nc),
        in_specs=[pl.BlockSpec((1, C_BLK, H, W), lambda b, c: (b, c, 0, 0))],
        out_specs=[
            pl.BlockSpec((1, H - 1, W), lambda b, c: (b, 0, 0)),
            pl.BlockSpec((1, H, W - 1), lambda b, c: (b, 0, 0)),
        ],
        out_shape=[
            jax.ShapeDtypeStruct((B, H - 1, W), jnp.float32),
            jax.ShapeDtypeStruct((B, H, W - 1), jnp.float32),
        ],
        scratch_shapes=[
            pltpu.VMEM((H - 1, W), jnp.float32),
            pltpu.VMEM((H, W - 1), jnp.float32),
        ],
    )(guide)


def _make_uf_kernel(B):
    mesh = plsc.VectorSubcoreMesh(core_axis_name="c", subcore_axis_name="s")

    @functools.partial(
        pl.kernel,
        out_type=jax.ShapeDtypeStruct((B, 2 * V), jnp.int32),
        mesh=mesh,
        scratch_types=[
            pltpu.VMEM((V,), jnp.int32),
            pltpu.VMEM((CE,), jnp.int32),
            pltpu.VMEM((2 * OUTCH,), jnp.int32),
        ],
    )
    def uf(suv_hbm, out_hbm, parent, ebuf, obuf):
        cid = lax.axis_index("c")
        sid = lax.axis_index("s")
        b = sid

        @pl.when((cid == 0) & (sid < B))
        def _():
            def init_body(i, carry):
                parent[pl.ds(i * 16, 16)] = lax.iota(jnp.int32, 16) + i * 16
                return carry

            lax.fori_loop(0, V // 16, init_body, jnp.int32(0))

            def find(x):
                def cond_f(y):
                    return parent[y] != y

                def body_f(y):
                    gp = parent[parent[y]]
                    parent[y] = gp
                    return gp

                return lax.while_loop(cond_f, body_f, x)

            def chunk_body(ch, cnt):
                def run():
                    pltpu.sync_copy(suv_hbm.at[b, pl.ds(ch * CE, CE)], ebuf)

                    def edge_body(e, cnt):
                        uvv = ebuf[e]
                        u = lax.shift_right_logical(uvv, 16)
                        v = jnp.bitwise_and(uvv, 0xFFFF)
                        ru = find(u)
                        rv = find(v)

                        def accept():
                            parent[ru] = rv
                            pos = jnp.bitwise_and(cnt, OUTCH - 1)
                            obuf[2 * pos] = u
                            obuf[2 * pos + 1] = v

                            @pl.when(pos == OUTCH - 1)
                            def _flush():
                                woff = (cnt + 1 - OUTCH) * 2
                                pltpu.sync_copy(
                                    obuf,
                                    out_hbm.at[b, pl.ds(pl.multiple_of(woff, 2 * OUTCH),
                                                        2 * OUTCH)],
                                )

                            return cnt + 1

                        return lax.cond(ru != rv, accept, lambda: cnt)

                    return lax.fori_loop(0, CE, edge_body, cnt)

                return lax.cond(cnt < V - 1, run, lambda: cnt)

            lax.fori_loop(0, E_PAD // CE, chunk_body, jnp.int32(0))

            pltpu.sync_copy(
                obuf.at[pl.ds(0, TAIL_COPY_W)],
                out_hbm.at[b, pl.ds(2 * FULL_FLUSHES * OUTCH, TAIL_COPY_W)],
            )

    return uf


def kernel(guide_in):
    B = guide_in.shape[0]
    wv, wh = _edge_weights(guide_in)
    keys = jnp.concatenate([wv.reshape(B, -1), wh.reshape(B, -1)], axis=1)
    uvb = jnp.broadcast_to(jnp.asarray(_UV), (B, E))
    _, suv = lax.sort((keys, uvb), dimension=1, num_keys=1, is_stable=True)
    suv = jnp.pad(suv, ((0, 0), (0, E_PAD - E)))
    flat = _make_uf_kernel(B)(suv)
    return flat[:, : 2 * (V - 1)].reshape(B, V - 1, 2)


# speedup vs baseline: 37.1271x; 37.1271x over previous
import functools

import jax
import jax.numpy as jnp
import numpy as np
from jax import lax
from jax.experimental import pallas as pl
from jax.experimental.pallas import tpu as pltpu
from jax.experimental.pallas import tpu_sc as plsc

H = 224
W = 224
V = H * W
E = (H - 1) * W + H * (W - 1)
CE = 2048
E_PAD = ((E + CE - 1) // CE) * CE
OUTCH = 4096
FULL_FLUSHES = (V - 1) // OUTCH
TAIL_ROWS = (V - 1) - FULL_FLUSHES * OUTCH
TAIL_COPY_W = 2 * (TAIL_ROWS + 1)
C_BLK = 32


def _edge_uv_const():
    r = np.arange(V, dtype=np.int64).reshape(H, W)
    u = np.concatenate([r[:-1].reshape(-1), r[:, :-1].reshape(-1)])
    v = np.concatenate([r[1:].reshape(-1), r[:, 1:].reshape(-1)])
    uv = (u << 16) | v
    return uv.astype(np.uint32).view(np.int32)


_UV = _edge_uv_const()


def _weights_body(x_ref, wv_ref, wh_ref, accv, acch):
    c_idx = pl.program_id(1)
    bv = None
    bh = None
    for c in range(C_BLK):
        xc = x_ref[0, c]
        dv = xc[1:, :] - xc[:-1, :]
        dh = xc[:, 1:] - xc[:, :-1]
        bv = dv * dv if bv is None else bv + dv * dv
        bh = dh * dh if bh is None else bh + dh * dh
    accv[...] = jnp.where(c_idx == 0, bv, accv[...] + bv)
    acch[...] = jnp.where(c_idx == 0, bh, acch[...] + bh)

    @pl.when(c_idx == pl.num_programs(1) - 1)
    def _():
        wv_ref[0] = accv[...]
        wh_ref[0] = acch[...]


def _edge_weights(guide):
    B, C, _, _ = guide.shape
    nc = C // C_BLK
    return pl.pallas_call(
        _weights_body,
        grid=(B, nc),
        in_specs=[pl.BlockSpec((1, C_BLK, H, W), lambda b, c: (b, c, 0, 0))],
        out_specs=[
            pl.BlockSpec((1, H - 1, W), lambda b, c: (b, 0, 0)),
            pl.BlockSpec((1, H, W - 1), lambda b, c: (b, 0, 0)),
        ],
        out_shape=[
            jax.ShapeDtypeStruct((B, H - 1, W), jnp.float32),
            jax.ShapeDtypeStruct((B, H, W - 1), jnp.float32),
        ],
        scratch_shapes=[
            pltpu.VMEM((H - 1, W), jnp.float32),
            pltpu.VMEM((H, W - 1), jnp.float32),
        ],
    )(guide)


def _make_uf_kernel(B):
    mesh = plsc.VectorSubcoreMesh(core_axis_name="c", subcore_axis_name="s")

    @functools.partial(
        pl.kernel,
        out_type=jax.ShapeDtypeStruct((B, 2 * V), jnp.int32),
        mesh=mesh,
        compiler_params=pltpu.CompilerParams(needs_layout_passes=False),
        scratch_types=[
            pltpu.VMEM((V + 16,), jnp.int32),
            pltpu.VMEM((CE + 16,), jnp.int32),
            pltpu.VMEM((2 * OUTCH + 16,), jnp.int32),
        ],
    )
    def uf(suv_hbm, out_hbm, parent, ebuf, obuf):
        cid = lax.axis_index("c")
        sid = lax.axis_index("s")
        b = sid

        @pl.when((cid == 0) & (sid < B))
        def _():
            lanes = lax.iota(jnp.int32, 16)
            lane0 = lanes == 0

            def init_body(i, carry):
                parent[pl.ds(i * 16, 16)] = lanes + i * 16
                return carry

            lax.fori_loop(0, V // 16, init_body, jnp.int32(0))

            def chunk_body(ch, cnt):
                pltpu.sync_copy(suv_hbm.at[b, pl.ds(ch * CE, CE)],
                                ebuf.at[pl.ds(0, CE)])

                def edge_body(e, cnt):
                    uvv = ebuf[pl.ds(e, 16)][0]
                    u = lax.shift_right_logical(uvv, 16)
                    v = jnp.bitwise_and(uvv, 0xFFFF)
                    pk = jnp.where(lane0, u, v)
                    for _ in range(16):
                        pk = plsc.load_gather(parent,
                                              [jnp.bitwise_and(pk, 0xFFFF)])
                    pku = pk[0]
                    pkv = pk[1]
                    ru = jnp.bitwise_and(pku, 0xFFFF)
                    rv = jnp.bitwise_and(pkv, 0xFFFF)
                    rank_u = lax.shift_right_logical(pku, 16)
                    rank_v = lax.shift_right_logical(pkv, 16)
                    take = ru != rv
                    u_lo = rank_u < rank_v
                    lo = jnp.where(u_lo, ru, rv)
                    hi = jnp.where(u_lo, rv, ru)
                    rank_lo = jnp.where(u_lo, rank_u, rank_v)
                    rank_hi = jnp.where(u_lo, rank_v, rank_u)
                    plsc.store_scatter(
                        parent, [jnp.full((16,), lo, jnp.int32)],
                        jnp.full((16,), lax.shift_left(rank_lo, 16) | hi,
                                 jnp.int32),
                        mask=lane0 & take)
                    plsc.store_scatter(
                        parent, [jnp.full((16,), hi, jnp.int32)],
                        jnp.full((16,), lax.shift_left(rank_hi + 1, 16) | hi,
                                 jnp.int32),
                        mask=lane0 & (take & (rank_u == rank_v)))
                    pos = jnp.bitwise_and(cnt, OUTCH - 1)
                    obuf[pl.ds(2 * pos, 16)] = jnp.where(lane0, u, v)

                    @pl.when(take & (pos == OUTCH - 1))
                    def _flush():
                        woff = (cnt + 1 - OUTCH) * 2
                        pltpu.sync_copy(
                            obuf.at[pl.ds(0, 2 * OUTCH)],
                            out_hbm.at[b, pl.ds(pl.multiple_of(woff, 2 * OUTCH),
                                                2 * OUTCH)],
                        )

                    return cnt + take.astype(jnp.int32)

                return lax.fori_loop(0, CE, edge_body, cnt)

            lax.fori_loop(0, E_PAD // CE, chunk_body, jnp.int32(0))

            pltpu.sync_copy(
                obuf.at[pl.ds(0, TAIL_COPY_W)],
                out_hbm.at[b, pl.ds(2 * FULL_FLUSHES * OUTCH, TAIL_COPY_W)],
            )

    return uf


def kernel(guide_in):
    B = guide_in.shape[0]
    sv, sh = _edge_weights(guide_in)
    wv = jnp.sqrt(sv) + 1.0
    wh = jnp.sqrt(sh) + 1.0
    keys = jnp.concatenate([wv.reshape(B, -1), wh.reshape(B, -1)], axis=1)
    uvb = jnp.broadcast_to(jnp.asarray(_UV), (B, E))
    _, suv = lax.sort((keys, uvb), dimension=1, num_keys=1, is_stable=True)
    suv = jnp.pad(suv, ((0, 0), (0, E_PAD - E)))
    flat = _make_uf_kernel(B)(suv)
    return flat[:, : 2 * (V - 1)].reshape(B, V - 1, 2)


# speedup vs baseline: 70.2904x; 1.8932x over previous
import functools

import jax
import jax.numpy as jnp
import numpy as np
from jax import lax
from jax.experimental import pallas as pl
from jax.experimental.pallas import tpu as pltpu
from jax.experimental.pallas import tpu_sc as plsc

H = 224
W = 224
V = H * W
E = (H - 1) * W + H * (W - 1)
CE = 2048
E_PAD = ((E + CE - 1) // CE) * CE
OUTCH = 4096
FULL_FLUSHES = (V - 1) // OUTCH
TAIL_ROWS = (V - 1) - FULL_FLUSHES * OUTCH
TAIL_COPY_W = 2 * (TAIL_ROWS + 1)
C_BLK = 32


def _edge_uv_const():
    r = np.arange(V, dtype=np.int64).reshape(H, W)
    u = np.concatenate([r[:-1].reshape(-1), r[:, :-1].reshape(-1)])
    v = np.concatenate([r[1:].reshape(-1), r[:, 1:].reshape(-1)])
    uv = (u << 16) | v
    return uv.astype(np.uint32).view(np.int32)


_UV = _edge_uv_const()


def _weights_body(x_ref, wv_ref, wh_ref, accv, acch):
    c_idx = pl.program_id(1)
    bv = None
    bh = None
    for c in range(C_BLK):
        xc = x_ref[0, c]
        dv = xc[1:, :] - xc[:-1, :]
        dh = xc[:, 1:] - xc[:, :-1]
        bv = dv * dv if bv is None else bv + dv * dv
        bh = dh * dh if bh is None else bh + dh * dh
    accv[...] = jnp.where(c_idx == 0, bv, accv[...] + bv)
    acch[...] = jnp.where(c_idx == 0, bh, acch[...] + bh)

    @pl.when(c_idx == pl.num_programs(1) - 1)
    def _():
        wv_ref[0] = accv[...]
        wh_ref[0] = acch[...]


def _edge_weights(guide):
    B, C, _, _ = guide.shape
    nc = C // C_BLK
    return pl.pallas_call(
        _weights_body,
        grid=(B, nc),
        in_specs=[pl.BlockSpec((1, C_BLK, H, W), lambda b, c: (b, c, 0, 0))],
        out_specs=[
            pl.BlockSpec((1, H - 1, W), lambda b, c: (b, 0, 0)),
            pl.BlockSpec((1, H, W - 1), lambda b, c: (b, 0, 0)),
        ],
        out_shape=[
            jax.ShapeDtypeStruct((B, H - 1, W), jnp.float32),
            jax.ShapeDtypeStruct((B, H, W - 1), jnp.float32),
        ],
        scratch_shapes=[
            pltpu.VMEM((H - 1, W), jnp.float32),
            pltpu.VMEM((H, W - 1), jnp.float32),
        ],
    )(guide)


def _make_uf_kernel(B):
    mesh = plsc.VectorSubcoreMesh(core_axis_name="c", subcore_axis_name="s")

    @functools.partial(
        pl.kernel,
        out_type=jax.ShapeDtypeStruct((B, 2 * V), jnp.int32),
        mesh=mesh,
        compiler_params=pltpu.CompilerParams(needs_layout_passes=False),
        scratch_types=[
            pltpu.VMEM((V + 16,), jnp.int32),
            pltpu.VMEM((CE + 16,), jnp.int32),
            pltpu.VMEM((2 * OUTCH + 16,), jnp.int32),
        ],
    )
    def uf(suv_hbm, out_hbm, parent, ebuf, obuf):
        cid = lax.axis_index("c")
        sid = lax.axis_index("s")
        b = sid

        @pl.when((cid == 0) & (sid < B))
        def _():
            lanes = lax.iota(jnp.int32, 16)
            lane0 = lanes == 0

            def init_body(i, carry):
                parent[pl.ds(i * 16, 16)] = lanes + i * 16
                return carry

            lax.fori_loop(0, V // 16, init_body, jnp.int32(0))

            def chunk_body(ch, cnt):
                pltpu.sync_copy(suv_hbm.at[b, pl.ds(ch * CE, CE)],
                                ebuf.at[pl.ds(0, CE)])

                def edge_body(e, cnt):
                    uvv = ebuf[pl.ds(e, 16)][0]
                    u = lax.shift_right_logical(uvv, 16)
                    v = jnp.bitwise_and(uvv, 0xFFFF)
                    pk = jnp.where(lane0, u, v)
                    e_uv = plsc.load_gather(parent, [pk])
                    p1 = jnp.bitwise_and(e_uv, 0xFFFF)
                    p2 = jnp.bitwise_and(plsc.load_gather(parent, [p1]),
                                         0xFFFF)

                    def more(p):
                        for _ in range(14):
                            p = jnp.bitwise_and(
                                plsc.load_gather(parent, [p]), 0xFFFF)
                        return p

                    pk2 = lax.cond(jnp.all(p2 == p1), lambda p: p, more, p2)
                    pk = plsc.load_gather(parent, [pk2])
                    pku = pk[0]
                    pkv = pk[1]
                    comp_val = jnp.bitwise_or(
                        jnp.bitwise_and(e_uv, jnp.int32(-65536)),
                        jnp.bitwise_and(pk, 0xFFFF))
                    plsc.store_scatter(
                        parent, [jnp.where(lane0, u, v)], comp_val,
                        mask=lanes < 2)
                    ru = jnp.bitwise_and(pku, 0xFFFF)
                    rv = jnp.bitwise_and(pkv, 0xFFFF)
                    rank_u = lax.shift_right_logical(pku, 16)
                    rank_v = lax.shift_right_logical(pkv, 16)
                    take = ru != rv
                    u_lo = rank_u < rank_v
                    lo = jnp.where(u_lo, ru, rv)
                    hi = jnp.where(u_lo, rv, ru)
                    rank_lo = jnp.where(u_lo, rank_u, rank_v)
                    rank_hi = jnp.where(u_lo, rank_v, rank_u)
                    plsc.store_scatter(
                        parent,
                        [jnp.where(lane0, lo, hi)],
                        jnp.where(lane0,
                                  lax.shift_left(rank_lo, 16) | hi,
                                  lax.shift_left(rank_hi + 1, 16) | hi),
                        mask=take & (lane0 | ((lanes == 1)
                                             & (rank_u == rank_v))))
                    pos = jnp.bitwise_and(cnt, OUTCH - 1)
                    obuf[pl.ds(2 * pos, 16)] = jnp.where(lane0, u, v)

                    @pl.when(take & (pos == OUTCH - 1))
                    def _flush():
                        woff = (cnt + 1 - OUTCH) * 2
                        pltpu.sync_copy(
                            obuf.at[pl.ds(0, 2 * OUTCH)],
                            out_hbm.at[b, pl.ds(pl.multiple_of(woff, 2 * OUTCH),
                                                2 * OUTCH)],
                        )

                    return cnt + take.astype(jnp.int32)

                return lax.fori_loop(0, CE, edge_body, cnt)

            lax.fori_loop(0, E_PAD // CE, chunk_body, jnp.int32(0))

            pltpu.sync_copy(
                obuf.at[pl.ds(0, TAIL_COPY_W)],
                out_hbm.at[b, pl.ds(2 * FULL_FLUSHES * OUTCH, TAIL_COPY_W)],
            )

    return uf


def kernel(guide_in):
    B = guide_in.shape[0]
    sv, sh = _edge_weights(guide_in)
    wv = jnp.sqrt(sv) + 1.0
    wh = jnp.sqrt(sh) + 1.0
    keys = jnp.concatenate([wv.reshape(B, -1), wh.reshape(B, -1)], axis=1)
    uvb = jnp.broadcast_to(jnp.asarray(_UV), (B, E))
    _, suv = lax.sort((keys, uvb), dimension=1, num_keys=1, is_stable=True)
    suv = jnp.pad(suv, ((0, 0), (0, E_PAD - E)))
    flat = _make_uf_kernel(B)(suv)
    return flat[:, : 2 * (V - 1)].reshape(B, V - 1, 2)


# speedup vs baseline: 85.3375x; 1.2141x over previous
import functools

import jax
import jax.numpy as jnp
import numpy as np
from jax import lax
from jax.experimental import pallas as pl
from jax.experimental.pallas import tpu as pltpu
from jax.experimental.pallas import tpu_sc as plsc

H = 224
W = 224
V = H * W
E = (H - 1) * W + H * (W - 1)
CE = 2048
E_PAD = ((E + CE - 1) // CE) * CE
OUTCH = 4096
FULL_FLUSHES = (V - 1) // OUTCH
TAIL_ROWS = (V - 1) - FULL_FLUSHES * OUTCH
TAIL_COPY_W = 2 * (TAIL_ROWS + 1)
C_BLK = 32


def _edge_uv_const():
    r = np.arange(V, dtype=np.int64).reshape(H, W)
    u = np.concatenate([r[:-1].reshape(-1), r[:, :-1].reshape(-1)])
    v = np.concatenate([r[1:].reshape(-1), r[:, 1:].reshape(-1)])
    uv = (u << 16) | v
    return uv.astype(np.uint32).view(np.int32)


_UV = _edge_uv_const()


def _weights_body(x_ref, wv_ref, wh_ref, accv, acch):
    c_idx = pl.program_id(1)
    bv = None
    bh = None
    for c in range(C_BLK):
        xc = x_ref[0, c]
        dv = xc[1:, :] - xc[:-1, :]
        dh = xc[:, 1:] - xc[:, :-1]
        bv = dv * dv if bv is None else bv + dv * dv
        bh = dh * dh if bh is None else bh + dh * dh
    accv[...] = jnp.where(c_idx == 0, bv, accv[...] + bv)
    acch[...] = jnp.where(c_idx == 0, bh, acch[...] + bh)

    @pl.when(c_idx == pl.num_programs(1) - 1)
    def _():
        wv_ref[0] = accv[...]
        wh_ref[0] = acch[...]


def _edge_weights(guide):
    B, C, _, _ = guide.shape
    nc = C // C_BLK
    return pl.pallas_call(
        _weights_body,
        grid=(B, nc),
        in_specs=[pl.BlockSpec((1, C_BLK, H, W), lambda b, c: (b, c, 0, 0))],
        out_specs=[
            pl.BlockSpec((1, H - 1, W), lambda b, c: (b, 0, 0)),
            pl.BlockSpec((1, H, W - 1), lambda b, c: (b, 0, 0)),
        ],
        out_shape=[
            jax.ShapeDtypeStruct((B, H - 1, W), jnp.float32),
            jax.ShapeDtypeStruct((B, H, W - 1), jnp.float32),
        ],
        scratch_shapes=[
            pltpu.VMEM((H - 1, W), jnp.float32),
            pltpu.VMEM((H, W - 1), jnp.float32),
        ],
    )(guide)


def _make_uf_kernel(B):
    mesh = plsc.VectorSubcoreMesh(core_axis_name="c", subcore_axis_name="s")

    @functools.partial(
        pl.kernel,
        out_type=jax.ShapeDtypeStruct((B, 2 * V), jnp.int32),
        mesh=mesh,
        compiler_params=pltpu.CompilerParams(needs_layout_passes=False),
        scratch_types=[
            pltpu.VMEM((V + 16,), jnp.int32),
            pltpu.VMEM((CE + 16,), jnp.int32),
            pltpu.VMEM((4 * OUTCH + 16,), jnp.int32),
        ],
    )
    def uf(suv_hbm, out_hbm, parent, ebuf, obuf):
        cid = lax.axis_index("c")
        sid = lax.axis_index("s")
        b = sid

        @pl.when((cid == 0) & (sid < B))
        def _():
            lanes = lax.iota(jnp.int32, 16)
            lane0 = lanes == 0

            def init_body(i, carry):
                parent[pl.ds(i * 16, 16)] = lanes + i * 16
                return carry

            lax.fori_loop(0, V // 16, init_body, jnp.int32(0))

            zeros16 = jnp.zeros((16,), jnp.int32)
            ones16 = jnp.full((16,), 1, jnp.int32)

            gdn = lax.GatherDimensionNumbers(
                offset_dims=(), collapsed_slice_dims=(0,),
                start_index_map=(0,))

            def bcast(x, idx):
                return lax.gather(
                    x, idx[:, None], gdn, (1,),
                    mode=lax.GatherScatterMode.PROMISE_IN_BOUNDS)

            def chunk_body(ch, carry):
                cnt_vec, nfl = carry
                pltpu.sync_copy(suv_hbm.at[b, pl.ds(ch * CE, CE)],
                                ebuf.at[pl.ds(0, CE)])

                def group_body(g, cnt_vec):
                    evec = ebuf[pl.ds(pl.multiple_of(g * 16, 16), 16)]
                    for j in range(16):
                        uvv = bcast(evec, jnp.full((16,), j, jnp.int32))
                        u_b = lax.shift_right_logical(uvv, 16)
                        v_b = jnp.bitwise_and(uvv, 0xFFFF)
                        pk0 = jnp.where(lane0, u_b, v_b)
                        e_uv = plsc.load_gather(parent, [pk0])
                        p1 = jnp.bitwise_and(e_uv, 0xFFFF)
                        p2 = jnp.bitwise_and(plsc.load_gather(parent, [p1]),
                                             0xFFFF)

                        def more(p):
                            for _ in range(14):
                                p = jnp.bitwise_and(
                                    plsc.load_gather(parent, [p]), 0xFFFF)
                            return p

                        pk2 = lax.cond(jnp.all(p2 == p1), lambda p: p, more,
                                       p2)
                        rent = plsc.load_gather(parent, [pk2])
                        plsc.store_scatter(
                            parent, [pk0],
                            jnp.bitwise_or(
                                jnp.bitwise_and(e_uv, jnp.int32(-65536)),
                                jnp.bitwise_and(rent, 0xFFFF)),
                            mask=lanes < 2)
                        ridx = jnp.bitwise_and(rent, 0xFFFF)
                        rnk = lax.shift_right_logical(rent, 16)
                        ru_b = bcast(ridx, zeros16)
                        rv_b = bcast(ridx, ones16)
                        rank_u = bcast(rnk, zeros16)
                        rank_v = bcast(rnk, ones16)
                        take_b = ru_b != rv_b
                        u_lo = rank_u < rank_v
                        lo_b = jnp.where(u_lo, ru_b, rv_b)
                        hi_b = jnp.where(u_lo, rv_b, ru_b)
                        rank_lo = jnp.where(u_lo, rank_u, rank_v)
                        rank_hi = jnp.where(u_lo, rank_v, rank_u)
                        plsc.store_scatter(
                            parent,
                            [jnp.where(lane0, lo_b, hi_b)],
                            jnp.where(lane0,
                                      lax.shift_left(rank_lo, 16) | hi_b,
                                      lax.shift_left(rank_hi + 1, 16) | hi_b),
                            mask=take_b & (lane0 | ((lanes == 1)
                                                    & (rank_u == rank_v))))
                        pos = jnp.bitwise_and(cnt_vec, 2 * OUTCH - 1)
                        plsc.store_scatter(
                            obuf, [2 * pos + lanes],
                            jnp.where(lane0, u_b, v_b),
                            mask=take_b & (lanes < 2))
                        cnt_vec = cnt_vec + jnp.where(take_b, 1, 0)
                    return cnt_vec

                cnt_vec = lax.fori_loop(0, CE // 16, group_body, cnt_vec)
                cnt_s = cnt_vec[0]
                due = lax.shift_right_logical(cnt_s, 12)

                @pl.when(due > nfl)
                def _flush():
                    half = jnp.bitwise_and(nfl, 1)
                    pltpu.sync_copy(
                        obuf.at[pl.ds(pl.multiple_of(half * 2 * OUTCH,
                                                     2 * OUTCH), 2 * OUTCH)],
                        out_hbm.at[b, pl.ds(pl.multiple_of(nfl * 2 * OUTCH,
                                                           2 * OUTCH),
                                            2 * OUTCH)],
                    )

                nfl = jnp.where(due > nfl, nfl + 1, nfl)
                return (cnt_vec, nfl)

            lax.fori_loop(0, E_PAD // CE, chunk_body,
                          (jnp.zeros((16,), jnp.int32), jnp.int32(0)))

            pltpu.sync_copy(
                obuf.at[pl.ds(0, TAIL_COPY_W)],
                out_hbm.at[b, pl.ds(2 * FULL_FLUSHES * OUTCH, TAIL_COPY_W)],
            )

    return uf


def kernel(guide_in):
    B = guide_in.shape[0]
    sv, sh = _edge_weights(guide_in)
    wv = jnp.sqrt(sv) + 1.0
    wh = jnp.sqrt(sh) + 1.0
    keys = jnp.concatenate([wv.reshape(B, -1), wh.reshape(B, -1)], axis=1)
    uvb = jnp.broadcast_to(jnp.asarray(_UV), (B, E))
    _, suv = lax.sort((keys, uvb), dimension=1, num_keys=1, is_stable=True)
    suv = jnp.pad(suv, ((0, 0), (0, E_PAD - E)))
    flat = _make_uf_kernel(B)(suv)
    return flat[:, : 2 * (V - 1)].reshape(B, V - 1, 2)
